# baseline (speedup 1.0000x reference)
import functools

import jax
import jax.numpy as jnp
from jax import lax
from jax.experimental import pallas as pl
from jax.experimental.pallas import tpu as pltpu
from jax.experimental.pallas import tpu_sc as plsc

BATCH = 4096
HIST = 50
EMBED_DIM = 64
HIST_PAD = 56

NUM_CORES = 2
NUM_SUBCORES = 16
NUM_WORKERS = NUM_CORES * NUM_SUBCORES
BATCH_PER_WORKER = BATCH // NUM_WORKERS
IDX_ROWS_PER_WORKER = BATCH_PER_WORKER * HIST // 128
BATCHES_PER_CHUNK = 4
CHUNK = BATCHES_PER_CHUNK * HIST
N_CHUNKS = BATCH_PER_WORKER // BATCHES_PER_CHUNK
NBUF = 4


def _build():
    mesh = plsc.VectorSubcoreMesh(core_axis_name="c", subcore_axis_name="s")

    @functools.partial(
        pl.kernel,
        mesh=mesh,
        out_type=jax.ShapeDtypeStruct((BATCH, HIST_PAD, 128), jnp.float32),
        scratch_types=[
            pltpu.VMEM((1, BATCH_PER_WORKER * HIST), jnp.int32),
            pltpu.VMEM((NBUF, CHUNK, EMBED_DIM), jnp.float32),
            pltpu.SemaphoreType.DMA((NBUF,)),
        ],
        compiler_params=pltpu.CompilerParams(use_tc_tiling_on_sc=False),
    )
    def gather_kernel(idx_hbm, table_hbm, out_hbm, idx_v, rows_v, sems):
        wid = lax.axis_index("s") * NUM_CORES + lax.axis_index("c")
        base = wid * BATCH_PER_WORKER

        pltpu.sync_copy(idx_hbm.at[pl.ds(wid, 1)], idx_v)

        def idx_at(c):
            return idx_v.at[0, pl.ds(c * CHUNK, CHUNK)]

        for b in range(NBUF):
            pltpu.async_copy(table_hbm.at[idx_at(b)], rows_v.at[b],
                             sems.at[b])

        def group(g, _):
            for b in range(NBUF):
                c = g * NBUF + b
                pltpu.make_async_copy(table_hbm.at[idx_at(b)],
                                      rows_v.at[b], sems.at[b]).wait()
                for j in range(BATCHES_PER_CHUNK):
                    pltpu.sync_copy(
                        rows_v.at[b].at[pl.ds(j * HIST, HIST)],
                        out_hbm.at[base + c * BATCHES_PER_CHUNK + j,
                                   pl.ds(0, HIST), pl.ds(0, EMBED_DIM)])
                nxt = c + NBUF

                @pl.when(nxt < N_CHUNKS)
                def _():
                    pltpu.async_copy(table_hbm.at[idx_at(nxt)],
                                     rows_v.at[b], sems.at[b])

            return ()

        lax.fori_loop(0, N_CHUNKS // NBUF, group, (), unroll=False)

    return gather_kernel


_gather = _build()


@jax.jit
def kernel(x, table):
    idx2d = x.reshape(NUM_WORKERS, BATCH_PER_WORKER * HIST)
    out = _gather(idx2d, table)
    return out[:, :HIST, :EMBED_DIM]


# speedup vs baseline: 1.0012x; 1.0012x over previous
import functools

import jax
import jax.numpy as jnp
from jax import lax
from jax.experimental import pallas as pl
from jax.experimental.pallas import tpu as pltpu
from jax.experimental.pallas import tpu_sc as plsc

BATCH = 4096
HIST = 50
EMBED_DIM = 64
HIST_PAD = 56

NUM_CORES = 2
NUM_SUBCORES = 16
NUM_WORKERS = NUM_CORES * NUM_SUBCORES
BATCH_PER_WORKER = BATCH // NUM_WORKERS
IDX_ROWS_PER_WORKER = BATCH_PER_WORKER * HIST // 128
BATCHES_PER_CHUNK = 4
CHUNK = BATCHES_PER_CHUNK * HIST
N_CHUNKS = BATCH_PER_WORKER // BATCHES_PER_CHUNK
NBUF = 8


def _build():
    mesh = plsc.VectorSubcoreMesh(core_axis_name="c", subcore_axis_name="s")

    @functools.partial(
        pl.kernel,
        mesh=mesh,
        out_type=jax.ShapeDtypeStruct((BATCH, HIST_PAD, 128), jnp.float32),
        scratch_types=[
            pltpu.VMEM((1, BATCH_PER_WORKER * HIST), jnp.int32),
            pltpu.VMEM((NBUF, CHUNK, EMBED_DIM), jnp.float32),
            pltpu.SemaphoreType.DMA((NBUF,)),
        ],
        compiler_params=pltpu.CompilerParams(use_tc_tiling_on_sc=False),
    )
    def gather_kernel(idx_hbm, table_hbm, out_hbm, idx_v, rows_v, sems):
        wid = lax.axis_index("s") * NUM_CORES + lax.axis_index("c")
        base = wid * BATCH_PER_WORKER

        pltpu.sync_copy(idx_hbm.at[pl.ds(wid, 1)], idx_v)

        def idx_at(c):
            return idx_v.at[0, pl.ds(c * CHUNK, CHUNK)]

        for b in range(NBUF):
            pltpu.async_copy(table_hbm.at[idx_at(b)], rows_v.at[b],
                             sems.at[b])

        def group(g, _):
            for b in range(NBUF):
                c = g * NBUF + b
                pltpu.make_async_copy(table_hbm.at[idx_at(b)],
                                      rows_v.at[b], sems.at[b]).wait()
                for j in range(BATCHES_PER_CHUNK):
                    pltpu.sync_copy(
                        rows_v.at[b].at[pl.ds(j * HIST, HIST)],
                        out_hbm.at[base + c * BATCHES_PER_CHUNK + j,
                                   pl.ds(0, HIST), pl.ds(0, EMBED_DIM)])
                nxt = c + NBUF

                @pl.when(nxt < N_CHUNKS)
                def _():
                    pltpu.async_copy(table_hbm.at[idx_at(nxt)],
                                     rows_v.at[b], sems.at[b])

            return ()

        lax.fori_loop(0, N_CHUNKS // NBUF, group, (), unroll=False)

    return gather_kernel


_gather = _build()


@jax.jit
def kernel(x, table):
    idx2d = x.reshape(NUM_WORKERS, BATCH_PER_WORKER * HIST)
    out = _gather(idx2d, table)
    return out[:, :HIST, :EMBED_DIM]


# speedup vs baseline: 1.0025x; 1.0014x over previous
import functools

import jax
import jax.numpy as jnp
from jax import lax
from jax.experimental import pallas as pl
from jax.experimental.pallas import tpu as pltpu
from jax.experimental.pallas import tpu_sc as plsc

BATCH = 4096
HIST = 50
EMBED_DIM = 64
HIST_PAD = 56

NUM_CORES = 2
NUM_SUBCORES = 16
NUM_WORKERS = NUM_CORES * NUM_SUBCORES
BATCH_PER_WORKER = BATCH // NUM_WORKERS
IDX_ROWS_PER_WORKER = BATCH_PER_WORKER * HIST // 128
BATCHES_PER_CHUNK = 8
CHUNK = BATCHES_PER_CHUNK * HIST
N_CHUNKS = BATCH_PER_WORKER // BATCHES_PER_CHUNK
NBUF = 4


def _build():
    mesh = plsc.VectorSubcoreMesh(core_axis_name="c", subcore_axis_name="s")

    @functools.partial(
        pl.kernel,
        mesh=mesh,
        out_type=jax.ShapeDtypeStruct((BATCH, HIST_PAD, 128), jnp.float32),
        scratch_types=[
            pltpu.VMEM((1, BATCH_PER_WORKER * HIST), jnp.int32),
            pltpu.VMEM((NBUF, CHUNK, EMBED_DIM), jnp.float32),
            pltpu.SemaphoreType.DMA((NBUF,)),
        ],
        compiler_params=pltpu.CompilerParams(use_tc_tiling_on_sc=False),
    )
    def gather_kernel(idx_hbm, table_hbm, out_hbm, idx_v, rows_v, sems):
        wid = lax.axis_index("s") * NUM_CORES + lax.axis_index("c")
        base = wid * BATCH_PER_WORKER

        pltpu.sync_copy(idx_hbm.at[pl.ds(wid, 1)], idx_v)

        def idx_at(c):
            return idx_v.at[0, pl.ds(c * CHUNK, CHUNK)]

        for b in range(NBUF):
            pltpu.async_copy(table_hbm.at[idx_at(b)], rows_v.at[b],
                             sems.at[b])

        def group(g, _):
            for b in range(NBUF):
                c = g * NBUF + b
                pltpu.make_async_copy(table_hbm.at[idx_at(b)],
                                      rows_v.at[b], sems.at[b]).wait()
                for j in range(BATCHES_PER_CHUNK):
                    pltpu.sync_copy(
                        rows_v.at[b].at[pl.ds(j * HIST, HIST)],
                        out_hbm.at[base + c * BATCHES_PER_CHUNK + j,
                                   pl.ds(0, HIST), pl.ds(0, EMBED_DIM)])
                nxt = c + NBUF

                @pl.when(nxt < N_CHUNKS)
                def _():
                    pltpu.async_copy(table_hbm.at[idx_at(nxt)],
                                     rows_v.at[b], sems.at[b])

            return ()

        lax.fori_loop(0, N_CHUNKS // NBUF, group, (), unroll=False)

    return gather_kernel


_gather = _build()


@jax.jit
def kernel(x, table):
    idx2d = x.reshape(NUM_WORKERS, BATCH_PER_WORKER * HIST)
    out = _gather(idx2d, table)
    return out[:, :HIST, :EMBED_DIM]
